# baseline (speedup 1.0000x reference)
import jax
import jax.numpy as jnp
from jax.experimental import pallas as pl
from jax.experimental.pallas import tpu as pltpu


def _make_kernel(channels, chunk, n_chunks):
    def _fused_kernel(ca_ref,
                      x_ref, e_ref, n_ref,
                      w1_ref,
                      b1_ref, temb_ref,
                      w2_ref, b2_ref,
                      out_ref):
        b = pl.program_id(0)

        c = ca_ref[b]
        s = jnp.sqrt(jnp.maximum(1.0 - c * c, 0.0))

        w1x = w1_ref[:channels].astype(jnp.bfloat16)
        w1n = w1_ref[channels:].astype(jnp.bfloat16)
        w2b = w2_ref[...].astype(jnp.bfloat16)
        bias1 = b1_ref[...] + c * temb_ref[...]
        bias2 = b2_ref[...]

        psum = jnp.zeros((channels,), jnp.float32)
        for i in range(n_chunks):
            sl = pl.ds(i * chunk, chunk)
            x = x_ref[0, sl, :]
            e = e_ref[0, sl, :]
            nz = n_ref[0, sl, :]

            x_noisy = c * (x - e) + s * nz

            h = (jnp.dot(x.astype(jnp.bfloat16), w1x,
                         preferred_element_type=jnp.float32)
                 + jnp.dot(x_noisy.astype(jnp.bfloat16), w1n,
                           preferred_element_type=jnp.float32))
            h = h + bias1
            h = h * jax.nn.sigmoid(h)

            out = (jnp.dot(h.astype(jnp.bfloat16), w2b,
                           preferred_element_type=jnp.float32)
                   + bias2)

            diff = nz - out
            psum = psum + jnp.sum(diff * diff, axis=0)

        out_ref[0, 0] = psum

    return _fused_kernel


def _pick_chunk(dhw, cap=4096):
    if dhw % 8 != 0:
        return dhw
    t = min(dhw, cap)
    while dhw % t != 0:
        t -= 8
    return t


def kernel(x, e, noise, sqrt_alpha, w1, b1, temb, w2, b2):
    B, C, D, H, W = x.shape
    DHW = D * H * W
    HID = w1.shape[1]

    chunk = _pick_chunk(DHW)
    n_chunks = DHW // chunk

    xt = jnp.transpose(x, (0, 2, 3, 4, 1)).reshape(B, DHW, C)
    et = jnp.transpose(e, (0, 2, 3, 4, 1)).reshape(B, DHW, C)
    nt = jnp.transpose(noise, (0, 2, 3, 4, 1)).reshape(B, DHW, C)

    grid_spec = pltpu.PrefetchScalarGridSpec(
        num_scalar_prefetch=1,
        grid=(B,),
        in_specs=[
            pl.BlockSpec((1, DHW, C), lambda b, ca: (b, 0, 0)),
            pl.BlockSpec((1, DHW, C), lambda b, ca: (b, 0, 0)),
            pl.BlockSpec((1, DHW, C), lambda b, ca: (b, 0, 0)),
            pl.BlockSpec((2 * C, HID), lambda b, ca: (0, 0)),
            pl.BlockSpec((1, HID), lambda b, ca: (0, 0)),
            pl.BlockSpec((1, HID), lambda b, ca: (0, 0)),
            pl.BlockSpec((HID, C), lambda b, ca: (0, 0)),
            pl.BlockSpec((1, C), lambda b, ca: (0, 0)),
        ],
        out_specs=pl.BlockSpec((1, 1, C), lambda b, ca: (b, 0, 0)),
    )

    partials = pl.pallas_call(
        _make_kernel(C, chunk, n_chunks),
        out_shape=jax.ShapeDtypeStruct((B, 1, C), jnp.float32),
        grid_spec=grid_spec,
        compiler_params=pltpu.CompilerParams(
            dimension_semantics=("arbitrary",),
            vmem_limit_bytes=64 * 1024 * 1024),
    )(sqrt_alpha, xt, et, nt, w1, b1, temb, w2, b2)

    return jnp.sum(partials) / (B * C * DHW)


# speedup vs baseline: 1.2712x; 1.2712x over previous
import jax
import jax.numpy as jnp
from jax.experimental import pallas as pl
from jax.experimental.pallas import tpu as pltpu


def _make_kernel(channels):
    def _fused_kernel(ca_ref,
                      x_ref, e_ref, n_ref,
                      w1_ref,
                      b1_ref, temb_ref,
                      w2_ref, b2_ref,
                      out_ref):
        b = pl.program_id(0)
        k = pl.program_id(1)

        c = ca_ref[b]
        s = jnp.sqrt(jnp.maximum(1.0 - c * c, 0.0))

        x = x_ref[0]
        e = e_ref[0]
        nz = n_ref[0]

        u = s * nz - c * e

        w1n = w1_ref[channels:].astype(jnp.bfloat16)
        wA = (w1_ref[:channels] + c * w1_ref[channels:]).astype(jnp.bfloat16)

        h = (jnp.dot(x.astype(jnp.bfloat16), wA,
                     preferred_element_type=jnp.float32)
             + jnp.dot(u.astype(jnp.bfloat16), w1n,
                       preferred_element_type=jnp.float32))
        h = h + (b1_ref[...] + c * temb_ref[...])
        th = jnp.tanh(0.5 * h)
        h = 0.5 * h * (1.0 + th)

        out = (jnp.dot(h.astype(jnp.bfloat16),
                       w2_ref[...].astype(jnp.bfloat16),
                       preferred_element_type=jnp.float32)
               + b2_ref[...])

        diff = nz - out
        psum = jnp.sum(diff * diff, axis=0)

        @pl.when(k == 0)
        def _():
            out_ref[0, 0] = jnp.zeros_like(psum)

        out_ref[0, 0] = out_ref[0, 0] + psum

    return _fused_kernel


def _pick_tile(dhw, cap=8192):
    if dhw % 8 != 0:
        return dhw
    t = min(dhw, cap)
    while dhw % t != 0:
        t -= 8
    return t


def kernel(x, e, noise, sqrt_alpha, w1, b1, temb, w2, b2):
    B, C, D, H, W = x.shape
    DHW = D * H * W
    HID = w1.shape[1]

    S = _pick_tile(DHW)
    n_tiles = DHW // S

    xt = jnp.transpose(x, (0, 2, 3, 4, 1)).reshape(B, DHW, C)
    et = jnp.transpose(e, (0, 2, 3, 4, 1)).reshape(B, DHW, C)
    nt = jnp.transpose(noise, (0, 2, 3, 4, 1)).reshape(B, DHW, C)

    grid_spec = pltpu.PrefetchScalarGridSpec(
        num_scalar_prefetch=1,
        grid=(B, n_tiles),
        in_specs=[
            pl.BlockSpec((1, S, C), lambda b, k, ca: (b, k, 0)),
            pl.BlockSpec((1, S, C), lambda b, k, ca: (b, k, 0)),
            pl.BlockSpec((1, S, C), lambda b, k, ca: (b, k, 0)),
            pl.BlockSpec((2 * C, HID), lambda b, k, ca: (0, 0)),
            pl.BlockSpec((1, HID), lambda b, k, ca: (0, 0)),
            pl.BlockSpec((1, HID), lambda b, k, ca: (0, 0)),
            pl.BlockSpec((HID, C), lambda b, k, ca: (0, 0)),
            pl.BlockSpec((1, C), lambda b, k, ca: (0, 0)),
        ],
        out_specs=pl.BlockSpec((1, 1, C), lambda b, k, ca: (b, 0, 0)),
    )

    partials = pl.pallas_call(
        _make_kernel(C),
        out_shape=jax.ShapeDtypeStruct((B, 1, C), jnp.float32),
        grid_spec=grid_spec,
        compiler_params=pltpu.CompilerParams(
            dimension_semantics=("arbitrary", "arbitrary"),
            vmem_limit_bytes=64 * 1024 * 1024),
    )(sqrt_alpha, xt, et, nt, w1, b1, temb, w2, b2)

    return jnp.sum(partials) / (B * C * DHW)
